# Initial kernel scaffold; baseline (speedup 1.0000x reference)
import jax
import jax.numpy as jnp
from jax.experimental import pallas as pl


def kernel(src, indices):
    raise NotImplementedError("write your pallas kernel here")


# speedup vs baseline: 1.4079x; 1.4079x over previous
import jax
import jax.numpy as jnp
from jax import lax
from jax.experimental import pallas as pl
from jax.experimental.pallas import tpu as pltpu
from jax.experimental.pallas import tpu_sc as plsc

N_ROWS = 4096
N_COLS = 4096
N_OUT = 4096
L = 16
NC = 2
NS = 16
NW = NC * NS
ROWS_PER_W = N_ROWS // NW
RB = 8
N_BLOCKS = ROWS_PER_W // RB
J_CHUNKS = N_OUT // L


def _sc_body(src_hbm, pos_hbm, out_hbm,
             posf, cidx, w0t, w1t, w2t, w3t, inb, outb):
    wid = lax.axis_index("s") * NC + lax.axis_index("c")

    pltpu.sync_copy(pos_hbm, posf)

    def wchunk(jb, _):
        sl = pl.ds(jb * L, L)
        t = posf[sl]
        i = t.astype(jnp.int32)
        f = t - i.astype(jnp.float32)
        f2 = f * f
        f3 = f2 * f
        cidx[sl] = i - 1
        w0t[sl] = -0.5 * f + f2 - 0.5 * f3
        w1t[sl] = 1.0 - 2.5 * f2 + 1.5 * f3
        w2t[sl] = 0.5 * f + 2.0 * f2 - 1.5 * f3
        w3t[sl] = -0.5 * f2 + 0.5 * f3
        return 0

    lax.fori_loop(0, J_CHUNKS, wchunk, 0)

    def rblock(b, _):
        base = (wid * ROWS_PER_W + b * RB) * N_COLS
        pltpu.sync_copy(src_hbm.at[pl.ds(base, RB * N_COLS)], inb)

        def jchunk(jb, _):
            sl = pl.ds(jb * L, L)
            cc = cidx[sl]
            c1 = cc + 1
            c2 = cc + 2
            c3 = jnp.minimum(cc + 3, N_COLS - 1)
            u0 = w0t[sl]
            u1 = w1t[sl]
            u2 = w2t[sl]
            u3 = w3t[sl]
            for r in range(RB):
                roff = r * N_COLS
                g0 = plsc.load_gather(inb, [cc + roff])
                g1 = plsc.load_gather(inb, [c1 + roff])
                g2 = plsc.load_gather(inb, [c2 + roff])
                g3 = plsc.load_gather(inb, [c3 + roff])
                outb[pl.ds(r * N_OUT + jb * L, L)] = (
                    u0 * g0 + u1 * g1 + u2 * g2 + u3 * g3)
            return 0

        lax.fori_loop(0, J_CHUNKS, jchunk, 0)
        pltpu.sync_copy(outb, out_hbm.at[pl.ds(base, RB * N_OUT)])
        return 0

    lax.fori_loop(0, N_BLOCKS, rblock, 0)


@jax.jit
def kernel(src, indices):
    mesh = plsc.VectorSubcoreMesh(core_axis_name="c", subcore_axis_name="s",
                                  num_cores=NC, num_subcores=NS)
    run = pl.kernel(
        _sc_body,
        out_type=jax.ShapeDtypeStruct((N_ROWS * N_OUT,), jnp.float32),
        mesh=mesh,
        compiler_params=pltpu.CompilerParams(needs_layout_passes=False),
        scratch_types=[
            pltpu.VMEM((N_OUT,), jnp.float32),
            pltpu.VMEM((N_OUT,), jnp.int32),
            pltpu.VMEM((N_OUT,), jnp.float32),
            pltpu.VMEM((N_OUT,), jnp.float32),
            pltpu.VMEM((N_OUT,), jnp.float32),
            pltpu.VMEM((N_OUT,), jnp.float32),
            pltpu.VMEM((RB * N_COLS,), jnp.float32),
            pltpu.VMEM((RB * N_OUT,), jnp.float32),
        ],
    )
    out_flat = run(src.reshape(N_ROWS * N_COLS), indices)
    return out_flat.reshape(N_ROWS, N_OUT)


# speedup vs baseline: 1.5966x; 1.1341x over previous
import jax
import jax.numpy as jnp
from jax import lax
from jax.experimental import pallas as pl
from jax.experimental.pallas import tpu as pltpu
from jax.experimental.pallas import tpu_sc as plsc

N_ROWS = 4096
N_COLS = 4096
N_OUT = 4096
L = 16
NC = 2
NS = 16
NW = NC * NS
ROWS_PER_W = N_ROWS // NW
RB = 8
N_BLOCKS = ROWS_PER_W // RB
J_CHUNKS = N_OUT // L


def _sc_body(src_hbm, pos_hbm, out_hbm,
             posf, cidx, w0t, w1t, w2t, w3t, inb, outb):
    wid = lax.axis_index("s") * NC + lax.axis_index("c")

    pltpu.sync_copy(pos_hbm, posf)

    def wchunk(jb, _):
        sl = pl.ds(jb * L, L)
        t = posf[sl]
        i = t.astype(jnp.int32)
        f = t - i.astype(jnp.float32)
        f2 = f * f
        f3 = f2 * f
        cidx[sl] = i - 1
        w0t[sl] = -0.5 * f + f2 - 0.5 * f3
        w1t[sl] = 1.0 - 2.5 * f2 + 1.5 * f3
        w2t[sl] = 0.5 * f + 2.0 * f2 - 1.5 * f3
        w3t[sl] = -0.5 * f2 + 0.5 * f3
        return 0

    lax.fori_loop(0, J_CHUNKS, wchunk, 0)

    def rblock(b, _):
        base = (wid * ROWS_PER_W + b * RB) * N_COLS
        pltpu.sync_copy(src_hbm.at[pl.ds(base, RB * N_COLS)], inb)

        @plsc.parallel_loop(0, J_CHUNKS, 1, unroll=4)
        def jchunk(jb):
            sl = pl.ds(jb * L, L)
            cc = cidx[sl]
            c1 = cc + 1
            c2 = cc + 2
            c3 = jnp.minimum(cc + 3, N_COLS - 1)
            u0 = w0t[sl]
            u1 = w1t[sl]
            u2 = w2t[sl]
            u3 = w3t[sl]
            for r in range(RB):
                roff = r * N_COLS
                g0 = plsc.load_gather(inb, [cc + roff])
                g1 = plsc.load_gather(inb, [c1 + roff])
                g2 = plsc.load_gather(inb, [c2 + roff])
                g3 = plsc.load_gather(inb, [c3 + roff])
                outb[pl.ds(r * N_OUT + jb * L, L)] = (
                    u0 * g0 + u1 * g1 + u2 * g2 + u3 * g3)
        pltpu.sync_copy(outb, out_hbm.at[pl.ds(base, RB * N_OUT)])
        return 0

    lax.fori_loop(0, N_BLOCKS, rblock, 0)


@jax.jit
def kernel(src, indices):
    mesh = plsc.VectorSubcoreMesh(core_axis_name="c", subcore_axis_name="s",
                                  num_cores=NC, num_subcores=NS)
    run = pl.kernel(
        _sc_body,
        out_type=jax.ShapeDtypeStruct((N_ROWS * N_OUT,), jnp.float32),
        mesh=mesh,
        compiler_params=pltpu.CompilerParams(needs_layout_passes=False),
        scratch_types=[
            pltpu.VMEM((N_OUT,), jnp.float32),
            pltpu.VMEM((N_OUT,), jnp.int32),
            pltpu.VMEM((N_OUT,), jnp.float32),
            pltpu.VMEM((N_OUT,), jnp.float32),
            pltpu.VMEM((N_OUT,), jnp.float32),
            pltpu.VMEM((N_OUT,), jnp.float32),
            pltpu.VMEM((RB * N_COLS,), jnp.float32),
            pltpu.VMEM((RB * N_OUT,), jnp.float32),
        ],
    )
    out_flat = run(src.reshape(N_ROWS * N_COLS), indices)
    return out_flat.reshape(N_ROWS, N_OUT)


# speedup vs baseline: 2.1222x; 1.3292x over previous
import jax
import jax.numpy as jnp
from jax import lax
from jax.experimental import pallas as pl
from jax.experimental.pallas import tpu as pltpu
from jax.experimental.pallas import tpu_sc as plsc

N_ROWS = 4096
N_COLS = 4096
N_OUT = 4096
L = 16
NC = 2
NS = 16
NW = NC * NS
ROWS_PER_W = N_ROWS // NW
RB = 4
N_BLOCKS = ROWS_PER_W // RB
N_HALF = N_BLOCKS // 2
J_CHUNKS = N_OUT // L
BLK = RB * N_COLS


def _sc_body(src_hbm, pos_hbm, out_hbm,
             posf, cidx, w0t, w1t, w2t, w3t,
             inb0, inb1, outb0, outb1,
             si0, si1, so0, so1):
    wid = lax.axis_index("s") * NC + lax.axis_index("c")
    base = wid * ROWS_PER_W * N_COLS

    def in_slice(b):
        return src_hbm.at[pl.ds(base + b * BLK, BLK)]

    def out_slice(b):
        return out_hbm.at[pl.ds(base + b * BLK, BLK)]

    pltpu.async_copy(in_slice(0), inb0, si0)

    pltpu.sync_copy(pos_hbm, posf)

    def wchunk(jb, _):
        sl = pl.ds(jb * L, L)
        t = posf[sl]
        i = t.astype(jnp.int32)
        f = t - i.astype(jnp.float32)
        f2 = f * f
        f3 = f2 * f
        cidx[sl] = i - 1
        w0t[sl] = -0.5 * f + f2 - 0.5 * f3
        w1t[sl] = 1.0 - 2.5 * f2 + 1.5 * f3
        w2t[sl] = 0.5 * f + 2.0 * f2 - 1.5 * f3
        w3t[sl] = -0.5 * f2 + 0.5 * f3
        return 0

    lax.fori_loop(0, J_CHUNKS, wchunk, 0)

    def compute(inb, outb):
        @plsc.parallel_loop(0, J_CHUNKS, 1, unroll=4)
        def jchunk(jb):
            sl = pl.ds(jb * L, L)
            cc = cidx[sl]
            c1 = cc + 1
            c2 = cc + 2
            c3 = jnp.minimum(cc + 3, N_COLS - 1)
            u0 = w0t[sl]
            u1 = w1t[sl]
            u2 = w2t[sl]
            u3 = w3t[sl]
            for r in range(RB):
                roff = r * N_COLS
                g0 = plsc.load_gather(inb, [cc + roff])
                g1 = plsc.load_gather(inb, [c1 + roff])
                g2 = plsc.load_gather(inb, [c2 + roff])
                g3 = plsc.load_gather(inb, [c3 + roff])
                outb[pl.ds(r * N_OUT + jb * L, L)] = (
                    u0 * g0 + u1 * g1 + u2 * g2 + u3 * g3)

    def hblock(h, _):
        b0 = 2 * h
        b1 = b0 + 1
        pltpu.async_copy(in_slice(b1), inb1, si1)

        pltpu.make_async_copy(in_slice(b0), inb0, si0).wait()

        @pl.when(h > 0)
        def _():
            pltpu.make_async_copy(outb0, out_slice(b0), so0).wait()

        compute(inb0, outb0)
        pltpu.async_copy(outb0, out_slice(b0), so0)

        @pl.when(h < N_HALF - 1)
        def _():
            pltpu.async_copy(in_slice(b0 + 2), inb0, si0)

        pltpu.make_async_copy(in_slice(b1), inb1, si1).wait()

        @pl.when(h > 0)
        def _():
            pltpu.make_async_copy(outb1, out_slice(b1), so1).wait()

        compute(inb1, outb1)
        pltpu.async_copy(outb1, out_slice(b1), so1)
        return 0

    lax.fori_loop(0, N_HALF, hblock, 0)

    pltpu.make_async_copy(outb0, out_slice(N_BLOCKS - 2), so0).wait()
    pltpu.make_async_copy(outb1, out_slice(N_BLOCKS - 1), so1).wait()


@jax.jit
def kernel(src, indices):
    mesh = plsc.VectorSubcoreMesh(core_axis_name="c", subcore_axis_name="s",
                                  num_cores=NC, num_subcores=NS)
    run = pl.kernel(
        _sc_body,
        out_type=jax.ShapeDtypeStruct((N_ROWS * N_OUT,), jnp.float32),
        mesh=mesh,
        compiler_params=pltpu.CompilerParams(needs_layout_passes=False),
        scratch_types=[
            pltpu.VMEM((N_OUT,), jnp.float32),
            pltpu.VMEM((N_OUT,), jnp.int32),
            pltpu.VMEM((N_OUT,), jnp.float32),
            pltpu.VMEM((N_OUT,), jnp.float32),
            pltpu.VMEM((N_OUT,), jnp.float32),
            pltpu.VMEM((N_OUT,), jnp.float32),
            pltpu.VMEM((BLK,), jnp.float32),
            pltpu.VMEM((BLK,), jnp.float32),
            pltpu.VMEM((BLK,), jnp.float32),
            pltpu.VMEM((BLK,), jnp.float32),
            pltpu.SemaphoreType.DMA,
            pltpu.SemaphoreType.DMA,
            pltpu.SemaphoreType.DMA,
            pltpu.SemaphoreType.DMA,
        ],
    )
    out_flat = run(src.reshape(N_ROWS * N_COLS), indices)
    return out_flat.reshape(N_ROWS, N_OUT)


# speedup vs baseline: 4.0605x; 1.9133x over previous
import jax
import jax.numpy as jnp
from jax import lax
from jax.experimental import pallas as pl
from jax.experimental.pallas import tpu as pltpu
from jax.experimental.pallas import tpu_sc as plsc

N_ROWS = 4096
N_COLS = 4096
N_OUT = 4096
L = 16
NC = 2
NS = 16
NW = NC * NS
ROWS_PER_W = N_ROWS // NW
RB = 4
N_BLOCKS = ROWS_PER_W // RB
N_HALF = N_BLOCKS // 2
J_CHUNKS = N_OUT // L


def _sc_body(src_hbm, pos_hbm, out_hbm,
             posf, cidx, w0t, w1t, w2t, w3t,
             inb0, inb1, outb0, outb1,
             si0, si1, so0, so1):
    wid = lax.axis_index("s") * NC + lax.axis_index("c")
    row0 = wid * ROWS_PER_W

    def in_slice(b):
        return src_hbm.at[pl.ds(row0 + b * RB, RB)]

    def out_slice(b):
        return out_hbm.at[pl.ds(row0 + b * RB, RB)]

    pltpu.async_copy(in_slice(0), inb0, si0)

    pltpu.sync_copy(pos_hbm, posf)

    def wchunk(jb, _):
        sl = pl.ds(jb * L, L)
        t = posf[sl]
        i = t.astype(jnp.int32)
        f = t - i.astype(jnp.float32)
        f2 = f * f
        f3 = f2 * f
        cidx[sl] = i - 1
        w0t[sl] = -0.5 * f + f2 - 0.5 * f3
        w1t[sl] = 1.0 - 2.5 * f2 + 1.5 * f3
        w2t[sl] = 0.5 * f + 2.0 * f2 - 1.5 * f3
        w3t[sl] = -0.5 * f2 + 0.5 * f3
        return 0

    lax.fori_loop(0, J_CHUNKS, wchunk, 0)

    def compute(inb, outb):
        @plsc.parallel_loop(0, J_CHUNKS, 1, unroll=4)
        def jchunk(jb):
            sl = pl.ds(jb * L, L)
            cc = cidx[sl]
            c1 = cc + 1
            c2 = cc + 2
            c3 = jnp.minimum(cc + 3, N_COLS - 1)
            u0 = w0t[sl]
            u1 = w1t[sl]
            u2 = w2t[sl]
            u3 = w3t[sl]
            for r in range(RB):
                rv = jnp.full((L,), r, dtype=jnp.int32)
                g0 = plsc.load_gather(inb, [rv, cc])
                g1 = plsc.load_gather(inb, [rv, c1])
                g2 = plsc.load_gather(inb, [rv, c2])
                g3 = plsc.load_gather(inb, [rv, c3])
                outb[r, sl] = u0 * g0 + u1 * g1 + u2 * g2 + u3 * g3

    def hblock(h, _):
        b0 = 2 * h
        b1 = b0 + 1
        pltpu.async_copy(in_slice(b1), inb1, si1)

        pltpu.make_async_copy(in_slice(b0), inb0, si0).wait()

        @pl.when(h > 0)
        def _():
            pltpu.make_async_copy(outb0, out_slice(b0), so0).wait()

        compute(inb0, outb0)
        pltpu.async_copy(outb0, out_slice(b0), so0)

        @pl.when(h < N_HALF - 1)
        def _():
            pltpu.async_copy(in_slice(b0 + 2), inb0, si0)

        pltpu.make_async_copy(in_slice(b1), inb1, si1).wait()

        @pl.when(h > 0)
        def _():
            pltpu.make_async_copy(outb1, out_slice(b1), so1).wait()

        compute(inb1, outb1)
        pltpu.async_copy(outb1, out_slice(b1), so1)
        return 0

    lax.fori_loop(0, N_HALF, hblock, 0)

    pltpu.make_async_copy(outb0, out_slice(N_BLOCKS - 2), so0).wait()
    pltpu.make_async_copy(outb1, out_slice(N_BLOCKS - 1), so1).wait()


@jax.jit
def kernel(src, indices):
    mesh = plsc.VectorSubcoreMesh(core_axis_name="c", subcore_axis_name="s",
                                  num_cores=NC, num_subcores=NS)
    run = pl.kernel(
        _sc_body,
        out_type=jax.ShapeDtypeStruct((N_ROWS, N_OUT), jnp.float32),
        mesh=mesh,
        compiler_params=pltpu.CompilerParams(needs_layout_passes=False),
        scratch_types=[
            pltpu.VMEM((N_OUT,), jnp.float32),
            pltpu.VMEM((N_OUT,), jnp.int32),
            pltpu.VMEM((N_OUT,), jnp.float32),
            pltpu.VMEM((N_OUT,), jnp.float32),
            pltpu.VMEM((N_OUT,), jnp.float32),
            pltpu.VMEM((N_OUT,), jnp.float32),
            pltpu.VMEM((RB, N_COLS), jnp.float32),
            pltpu.VMEM((RB, N_COLS), jnp.float32),
            pltpu.VMEM((RB, N_OUT), jnp.float32),
            pltpu.VMEM((RB, N_OUT), jnp.float32),
            pltpu.SemaphoreType.DMA,
            pltpu.SemaphoreType.DMA,
            pltpu.SemaphoreType.DMA,
            pltpu.SemaphoreType.DMA,
        ],
    )
    return run(src, indices)
